# Initial kernel scaffold; baseline (speedup 1.0000x reference)
import jax
import jax.numpy as jnp
from jax.experimental import pallas as pl


def kernel(x, edge_index, W1, b1, W2, b2, W3, b3, gW1, gb1, gW2, gb2, linW, linb):
    raise NotImplementedError("write your pallas kernel here")


# speedup vs baseline: 6.0339x; 6.0339x over previous
import functools

import jax
import jax.numpy as jnp
from jax import lax
from jax.experimental import pallas as pl
from jax.experimental.pallas import tpu as pltpu
from jax.experimental.pallas import tpu_sc as plsc

B = 2
N = 10000
T = 64
F = 128
NTEC = 16
NCORE = 2
RPT = 640
NP = NTEC * RPT
SUB = 40
NRQ = RPT // SUB
EC = 56
K_HOPS = 39
BN = 256
NBLK = 40
NEG_SLOPE = 0.01


def _leaky(v):
    return jnp.where(v > 0, v, NEG_SLOPE * v)


def _sc_deg(col_a, ones_hbm, zrow_hbm):
    nchunk = col_a.shape[1]
    mesh = plsc.VectorSubcoreMesh(core_axis_name="c", subcore_axis_name="s")

    @functools.partial(
        pl.kernel,
        mesh=mesh,
        out_type=jax.ShapeDtypeStruct((NCORE * NP,), jnp.float32),
        scratch_types=[
            pltpu.VMEM((nchunk, EC), jnp.int32),
            pltpu.VMEM((EC,), jnp.float32),
            pltpu.VMEM((RPT,), jnp.float32),
            pltpu.VMEM_SHARED((NP,), jnp.float32),
        ],
    )
    def deg_kernel(col_hbm, ones_in, zrow_in, deg_out, col_v, ones_v, buf_v,
                   deg_sh):
        c = lax.axis_index("c")
        s = lax.axis_index("s")
        w = s * NCORE + c
        pltpu.sync_copy(col_hbm.at[w], col_v)
        pltpu.sync_copy(ones_in, ones_v)
        pltpu.sync_copy(zrow_in, buf_v)
        pltpu.sync_copy(buf_v, deg_sh.at[pl.ds(s * RPT, RPT)])
        plsc.subcore_barrier()

        def body(i, carry):
            pltpu.sync_copy(ones_v, deg_sh.at[col_v.at[i]], add=True)
            return carry

        lax.fori_loop(0, nchunk, body, 0)
        plsc.subcore_barrier()
        pltpu.sync_copy(deg_sh.at[pl.ds(s * RPT, RPT)], buf_v)
        pltpu.sync_copy(buf_v, deg_out.at[pl.ds(c * NP + s * RPT, RPT)])

    return deg_kernel(col_a, ones_hbm, zrow_hbm)


def _sc_hops(z0_flat, d2e, idx_b):
    nchunk = idx_b.shape[2]
    npair = nchunk // 2
    mesh = plsc.VectorSubcoreMesh(core_axis_name="c", subcore_axis_name="s")

    @functools.partial(
        pl.kernel,
        mesh=mesh,
        out_type=jax.ShapeDtypeStruct((B * NP, F), jnp.float32),
        scratch_types=[
            pltpu.VMEM((2, EC), jnp.int32),
            pltpu.VMEM((2, EC), jnp.int32),
            pltpu.VMEM((2, EC, F), jnp.float32),
            pltpu.VMEM((SUB, F), jnp.float32),
            pltpu.VMEM((SUB, F), jnp.float32),
            pltpu.VMEM_SHARED((NP, F), jnp.float32),
            pltpu.SemaphoreType.DMA,
            pltpu.SemaphoreType.DMA,
            pltpu.SemaphoreType.DMA,
            pltpu.SemaphoreType.DMA,
        ],
    )
    def hops_kernel(z0_hbm, d2e_hbm, idx_hbm, zout_hbm,
                    ibuf0, ibuf1, stag, sbuf, d2b, s_sh,
                    sem0, sem1, semi0, semi1):
        c = lax.axis_index("c")
        s = lax.axis_index("s")
        r0 = s * RPT

        def zero_sbuf():
            @plsc.parallel_loop(0, SUB * (F // 16), unroll=8)
            def _(u):
                sbuf[u // (F // 16), pl.ds((u % (F // 16)) * 16, 16)] = (
                    jnp.zeros((16,), jnp.float32))

        zero_sbuf()

        def initz_q(q, carry):
            pltpu.sync_copy(sbuf, s_sh.at[pl.ds(r0 + q * SUB, SUB)])
            return carry

        lax.fori_loop(0, NRQ, initz_q, 0)

        def init_q(q, carry):
            rq = c * NP + r0 + q * SUB
            pltpu.sync_copy(z0_hbm.at[pl.ds(rq, SUB)], sbuf)
            pltpu.sync_copy(sbuf, zout_hbm.at[pl.ds(rq, SUB)])
            return carry

        lax.fori_loop(0, NRQ, init_q, 0)
        plsc.subcore_barrier()

        def hop(h, carry):
            pltpu.sync_copy(idx_hbm.at[c, s, 0], ibuf0)
            pltpu.async_copy(zout_hbm.at[ibuf0.at[0]], stag.at[0], sem0)
            pltpu.async_copy(idx_hbm.at[c, s, 1], ibuf1, semi1)

            def chunk_pair(j, carry2):
                i1 = 2 * j + 1
                pltpu.make_async_copy(
                    zout_hbm.at[ibuf0.at[0]], stag.at[0], sem0).wait()
                pltpu.make_async_copy(
                    idx_hbm.at[c, s, i1], ibuf1, semi1).wait()
                pltpu.async_copy(
                    zout_hbm.at[ibuf1.at[0]], stag.at[1], sem1)
                pltpu.sync_copy(stag.at[0], s_sh.at[ibuf0.at[1]], add=True)

                @pl.when(j < npair - 1)
                def _():
                    pltpu.async_copy(idx_hbm.at[c, s, i1 + 1], ibuf0, semi0)

                pltpu.make_async_copy(
                    zout_hbm.at[ibuf1.at[0]], stag.at[1], sem1).wait()
                pltpu.sync_copy(stag.at[1], s_sh.at[ibuf1.at[1]], add=True)

                @pl.when(j < npair - 1)
                def _():
                    pltpu.make_async_copy(
                        idx_hbm.at[c, s, i1 + 1], ibuf0, semi0).wait()
                    pltpu.async_copy(
                        zout_hbm.at[ibuf0.at[0]], stag.at[0], sem0)
                    pltpu.async_copy(idx_hbm.at[c, s, i1 + 2], ibuf1, semi1)

                return carry2

            lax.fori_loop(0, npair, chunk_pair, 0)
            plsc.subcore_barrier()

            def scale_q(q, carry2):
                rq = r0 + q * SUB
                pltpu.sync_copy(s_sh.at[pl.ds(rq, SUB)], sbuf)
                pltpu.sync_copy(d2e_hbm.at[pl.ds(rq, SUB)], d2b)

                @plsc.parallel_loop(0, SUB * (F // 16), unroll=8)
                def _(u):
                    r = u // (F // 16)
                    cc = (u % (F // 16)) * 16
                    sbuf[r, pl.ds(cc, 16)] = (
                        sbuf[r, pl.ds(cc, 16)] * d2b[r, pl.ds(cc, 16)])

                pltpu.sync_copy(sbuf, zout_hbm.at[pl.ds(c * NP + rq, SUB)])
                zero_sbuf()
                pltpu.sync_copy(sbuf, s_sh.at[pl.ds(rq, SUB)])
                return carry2

            lax.fori_loop(0, NRQ, scale_q, 0)
            plsc.subcore_barrier()
            return carry

        lax.fori_loop(0, K_HOPS, hop, 0)

    return hops_kernel(z0_flat, d2e, idx_b)


def _tc_prep(degp):
    def body(dp_ref, dinv_ref, sdeg_ref, d2_ref):
        deg = dp_ref[0] + dp_ref[1]
        fid = (jax.lax.broadcasted_iota(jnp.int32, (NP // F, F), 0) * F
               + jax.lax.broadcasted_iota(jnp.int32, (NP // F, F), 1))
        mask = fid < N
        degs = jnp.maximum(deg, 1.0)
        dinv_ref[...] = jnp.where(mask, jax.lax.rsqrt(degs), 0.0)
        sdeg_ref[...] = jnp.where(mask, jnp.sqrt(degs), 0.0)
        d2_ref[...] = jnp.where(mask, 1.0 / degs, 0.0)

    shp = jax.ShapeDtypeStruct((NP // F, F), jnp.float32)
    return pl.pallas_call(body, out_shape=(shp, shp, shp))(degp)


def _tc_conv(x, w1r, w2t, w3t, b1, b2, b3):
    def body(x_ref, w1_ref, w2_ref, w3_ref, b1_ref, b2_ref, b3_ref, y_ref):
        xb = x_ref[0]
        x2 = xb.reshape(BN, T // 2, 2)
        h1 = jnp.zeros((BN, 30, F), jnp.float32)
        for tap in range(5):
            m = tap // 2
            par = tap % 2
            sl = x2[:, m:m + 30, par]
            h1 = h1 + sl[:, :, None] * w1_ref[tap][None, :, :]
        h1 = _leaky(h1 + b1_ref[0][None, None, :])

        h1r = h1.reshape(BN, 15, 2, F)
        o2 = jnp.zeros((BN * 13, F), jnp.float32)
        for tap in range(5):
            m = tap // 2
            par = tap % 2
            sl = h1r[:, m:m + 13, par, :].reshape(BN * 13, F)
            o2 = o2 + jnp.dot(sl, w2_ref[tap],
                              preferred_element_type=jnp.float32)
        h2 = _leaky(o2.reshape(BN, 13, F) + b2_ref[0][None, None, :])

        h2p = jnp.concatenate(
            [h2, jnp.zeros((BN, 1, F), jnp.float32)], axis=1)
        h2r = h2p.reshape(BN, 7, 2, F)
        o3 = jnp.zeros((BN * 5, F), jnp.float32)
        for tap in range(5):
            m = tap // 2
            par = tap % 2
            sl = h2r[:, m:m + 5, par, :].reshape(BN * 5, F)
            o3 = o3 + jnp.dot(sl, w3_ref[tap],
                              preferred_element_type=jnp.float32)
        h3 = o3.reshape(BN, 5, F) + b3_ref[0][None, None, :]
        y_ref[0] = jnp.max(h3, axis=1)

    grid = (B, NBLK)
    return pl.pallas_call(
        body,
        grid=grid,
        in_specs=[
            pl.BlockSpec((1, BN, T), lambda b, j: (b, j, 0)),
            pl.BlockSpec((5, 1, F), lambda b, j: (0, 0, 0)),
            pl.BlockSpec((5, F, F), lambda b, j: (0, 0, 0)),
            pl.BlockSpec((5, F, F), lambda b, j: (0, 0, 0)),
            pl.BlockSpec((1, F), lambda b, j: (0, 0)),
            pl.BlockSpec((1, F), lambda b, j: (0, 0)),
            pl.BlockSpec((1, F), lambda b, j: (0, 0)),
        ],
        out_specs=pl.BlockSpec((1, BN, F), lambda b, j: (b, j, 0)),
        out_shape=jax.ShapeDtypeStruct((B, N, F), jnp.float32),
    )(x, w1r, w2t, w3t, b1, b2, b3)


def _tc_scale_z0(y3, dinv3, d2_3):
    def body(y_ref, dinv_ref, d2_ref, z_ref, d2e_ref):
        j = pl.program_id(1)
        dv = dinv_ref[0, 0].reshape(BN, 1)
        rows = jax.lax.broadcasted_iota(jnp.int32, (BN, 1), 0) + j * BN
        z_ref[0] = jnp.where(rows < N, dv * y_ref[0], 0.0)
        d2e_ref[...] = jnp.broadcast_to(d2_ref[0, 0].reshape(BN, 1), (BN, F))

    return pl.pallas_call(
        body,
        grid=(B, NBLK),
        in_specs=[
            pl.BlockSpec((1, BN, F), lambda b, j: (b, j, 0)),
            pl.BlockSpec((1, 1, BN), lambda b, j: (j, 0, 0)),
            pl.BlockSpec((1, 1, BN), lambda b, j: (j, 0, 0)),
        ],
        out_specs=[
            pl.BlockSpec((1, BN, F), lambda b, j: (b, j, 0)),
            pl.BlockSpec((BN, F), lambda b, j: (j, 0)),
        ],
        out_shape=[
            jax.ShapeDtypeStruct((B, NP, F), jnp.float32),
            jax.ShapeDtypeStruct((NP, F), jnp.float32),
        ],
    )(y3, dinv3, d2_3)


def _tc_mix1(z39, sdeg3, dinv3, gw1t, gb1):
    def body(z_ref, sdeg_ref, dinv_ref, w_ref, b_ref, o_ref):
        sv = sdeg_ref[0, 0].reshape(BN, 1)
        dv = dinv_ref[0, 0].reshape(BN, 1)
        x39 = sv * z_ref[0]
        h = _leaky(jnp.dot(x39, w_ref[...],
                           preferred_element_type=jnp.float32)
                   + b_ref[0][None, :])
        o_ref[0] = dv * h

    return pl.pallas_call(
        body,
        grid=(B, NBLK),
        in_specs=[
            pl.BlockSpec((1, BN, F), lambda b, j: (b, j, 0)),
            pl.BlockSpec((1, 1, BN), lambda b, j: (j, 0, 0)),
            pl.BlockSpec((1, 1, BN), lambda b, j: (j, 0, 0)),
            pl.BlockSpec((F, F), lambda b, j: (0, 0)),
            pl.BlockSpec((1, F), lambda b, j: (0, 0)),
        ],
        out_specs=pl.BlockSpec((1, BN, F), lambda b, j: (b, j, 0)),
        out_shape=jax.ShapeDtypeStruct((B, NP, F), jnp.float32),
    )(z39, sdeg3, dinv3, gw1t, gb1)


def _tc_mix2(z39, sdeg3, gw2t, gb2):
    def body(z_ref, sdeg_ref, w_ref, b_ref, o_ref):
        j = pl.program_id(1)
        sv = sdeg_ref[0, 0].reshape(BN, 1)
        x39 = sv * z_ref[0]
        h = _leaky(jnp.dot(x39, w_ref[...],
                           preferred_element_type=jnp.float32)
                   + b_ref[0][None, :])
        rows = jax.lax.broadcasted_iota(jnp.int32, (BN, 1), 0) + j * BN
        hm = jnp.where(rows < N, h, 0.0)
        ps = jnp.sum(hm.reshape(BN // 8, 8, F), axis=0)[None]

        @pl.when(j == 0)
        def _():
            o_ref[...] = ps

        @pl.when(j != 0)
        def _():
            o_ref[...] = o_ref[...] + ps

    return pl.pallas_call(
        body,
        grid=(B, NBLK),
        in_specs=[
            pl.BlockSpec((1, BN, F), lambda b, j: (b, j, 0)),
            pl.BlockSpec((1, 1, BN), lambda b, j: (j, 0, 0)),
            pl.BlockSpec((F, F), lambda b, j: (0, 0)),
            pl.BlockSpec((1, F), lambda b, j: (0, 0)),
        ],
        out_specs=pl.BlockSpec((1, 8, F), lambda b, j: (b, 0, 0)),
        out_shape=jax.ShapeDtypeStruct((B, 8, F), jnp.float32),
    )(z39, sdeg3, gw2t, gb2)


def _tc_head(hsum, linwt_pad, linb_pad):
    def body(h_ref, w_ref, b_ref, o_ref):
        hs = jnp.sum(h_ref[...], axis=1)
        o_ref[...] = (jnp.dot(hs * (1.0 / N), w_ref[...],
                              preferred_element_type=jnp.float32)
                      + b_ref[...])

    return pl.pallas_call(
        body,
        out_shape=jax.ShapeDtypeStruct((B, F), jnp.float32),
    )(hsum, linwt_pad, linb_pad)


def kernel(x, edge_index, W1, b1, W2, b2, W3, b3, gW1, gb1, gW2, gb2,
           linW, linb):
    E = edge_index.shape[1]
    e_real = E + N
    epad = ((e_real + 32 * EC - 1) // (32 * EC)) * (32 * EC)
    npad_e = epad - e_real

    loop = jnp.arange(N, dtype=jnp.int32)
    pad_idx = N + (jnp.arange(npad_e, dtype=jnp.int32) % (NP - N))
    row_full = jnp.concatenate([edge_index[0], loop, pad_idx])
    col_full = jnp.concatenate([edge_index[1], loop, pad_idx])

    col_a = col_full.reshape(32, epad // (32 * EC), EC)
    row_1 = row_full.reshape(NTEC, epad // (NTEC * EC), EC)
    col_1 = col_full.reshape(NTEC, epad // (NTEC * EC), EC)
    idx_b = jnp.stack([
        jnp.stack([row_1, col_1], axis=2),
        jnp.stack([row_1 + NP, col_1], axis=2),
    ])

    ones_hbm = jnp.ones((EC,), jnp.float32)
    zrow_hbm = jnp.zeros((RPT,), jnp.float32)

    w1r = jnp.transpose(W1, (2, 1, 0))
    w2t = jnp.transpose(W2, (2, 1, 0))
    w3t = jnp.transpose(W3, (2, 1, 0))
    b1r = b1.reshape(1, F)
    b2r = b2.reshape(1, F)
    b3r = b3.reshape(1, F)
    gw1t = jnp.transpose(gW1)
    gw2t = jnp.transpose(gW2)
    linwt_pad = jnp.zeros((F, F), jnp.float32).at[:, :2].set(jnp.transpose(linW))
    linb_pad = jnp.zeros((B, F), jnp.float32).at[:, :2].set(linb[None, :])

    degp = _sc_deg(col_a, ones_hbm, zrow_hbm)
    degp3 = degp.reshape(B, NP // F, F)

    dinv2, sdeg2, d2_2 = _tc_prep(degp3)
    dinv3 = dinv2.reshape(NBLK, 1, BN)
    sdeg3 = sdeg2.reshape(NBLK, 1, BN)
    d2_3 = d2_2.reshape(NBLK, 1, BN)

    y3 = _tc_conv(x, w1r, w2t, w3t, b1r, b2r, b3r)

    z0, d2e = _tc_scale_z0(y3, dinv3, d2_3)
    z0 = z0.reshape(B * NP, F)

    z39a = _sc_hops(z0, d2e, idx_b)
    z0b = _tc_mix1(z39a.reshape(B, NP, F), sdeg3, dinv3, gw1t,
                   gb1.reshape(1, F))
    z39b = _sc_hops(z0b.reshape(B * NP, F), d2e, idx_b)
    hsum = _tc_mix2(z39b.reshape(B, NP, F), sdeg3, gw2t, gb2.reshape(1, F))
    outp = _tc_head(hsum, linwt_pad, linb_pad)

    out = outp[:, :2]
    y = y3.reshape(B * N, F)
    return (out, y)


# speedup vs baseline: 11.4679x; 1.9006x over previous
import functools

import jax
import jax.numpy as jnp
from jax import lax
from jax.experimental import pallas as pl
from jax.experimental.pallas import tpu as pltpu
from jax.experimental.pallas import tpu_sc as plsc

B = 2
N = 10000
T = 64
F = 128
NTEC = 16
NCORE = 2
RPT = 640
NP = NTEC * RPT
FC = 64
SUB = 80
NRQ = RPT // SUB
EC = 128
NBUF = 6
K_HOPS = 39
BN = 256
NBLK = 40
NEG_SLOPE = 0.01


def _leaky(v):
    return jnp.where(v > 0, v, NEG_SLOPE * v)


def _sc_deg(col_a, ones_hbm, zrow_hbm):
    nchunk = col_a.shape[1]
    mesh = plsc.VectorSubcoreMesh(core_axis_name="c", subcore_axis_name="s")

    @functools.partial(
        pl.kernel,
        mesh=mesh,
        out_type=jax.ShapeDtypeStruct((NCORE * NP,), jnp.float32),
        scratch_types=[
            pltpu.VMEM((nchunk, EC), jnp.int32),
            pltpu.VMEM((EC,), jnp.float32),
            pltpu.VMEM((RPT,), jnp.float32),
            pltpu.VMEM_SHARED((NP,), jnp.float32),
        ],
    )
    def deg_kernel(col_hbm, ones_in, zrow_in, deg_out, col_v, ones_v, buf_v,
                   deg_sh):
        c = lax.axis_index("c")
        s = lax.axis_index("s")
        w = s * NCORE + c
        pltpu.sync_copy(col_hbm.at[w], col_v)
        pltpu.sync_copy(ones_in, ones_v)
        pltpu.sync_copy(zrow_in, buf_v)
        pltpu.sync_copy(buf_v, deg_sh.at[pl.ds(s * RPT, RPT)])
        plsc.subcore_barrier()

        def body(i, carry):
            pltpu.sync_copy(ones_v, deg_sh.at[col_v.at[i]], add=True)
            return carry

        lax.fori_loop(0, nchunk, body, 0)
        plsc.subcore_barrier()
        pltpu.sync_copy(deg_sh.at[pl.ds(s * RPT, RPT)], buf_v)
        pltpu.sync_copy(buf_v, deg_out.at[pl.ds(c * NP + s * RPT, RPT)])

    return deg_kernel(col_a, ones_hbm, zrow_hbm)


def _sc_hops(z0a, z0b, d2e, row_b, col_b):
    nchunk = col_b.shape[1]
    nits = nchunk // NBUF
    mesh = plsc.VectorSubcoreMesh(core_axis_name="c", subcore_axis_name="s")
    zshape = jax.ShapeDtypeStruct((B * NP, FC), jnp.float32)

    @functools.partial(
        pl.kernel,
        mesh=mesh,
        out_type=(zshape, zshape),
        scratch_types=[
            pltpu.VMEM((nchunk, EC), jnp.int32),
            pltpu.VMEM((nchunk, EC), jnp.int32),
            pltpu.VMEM((NBUF, EC, FC), jnp.float32),
            pltpu.VMEM((SUB, FC), jnp.float32),
            pltpu.VMEM((SUB, FC), jnp.float32),
            pltpu.VMEM_SHARED((NP, FC), jnp.float32),
        ] + [pltpu.SemaphoreType.DMA] * (2 * NBUF),
        compiler_params=pltpu.CompilerParams(use_tc_tiling_on_sc=False),
    )
    def hops_kernel(z0a_hbm, z0b_hbm, d2e_hbm, row_hbm, col_hbm,
                    zouta_hbm, zoutb_hbm,
                    row_v, col_v, stag, sbuf, d2b, s_sh, *sems):
        semg = sems[:NBUF]
        semsc = sems[NBUF:]
        c = lax.axis_index("c")
        s = lax.axis_index("s")
        r0 = s * RPT
        pltpu.sync_copy(row_hbm.at[c, s], row_v)
        pltpu.sync_copy(col_hbm.at[s], col_v)

        def zero_sbuf():
            @plsc.parallel_loop(0, SUB * (FC // 16), unroll=8)
            def _(u):
                sbuf[u // (FC // 16), pl.ds((u % (FC // 16)) * 16, 16)] = (
                    jnp.zeros((16,), jnp.float32))

        zero_sbuf()

        def initz_q(q, carry):
            pltpu.sync_copy(sbuf, s_sh.at[pl.ds(r0 + q * SUB, SUB)])
            return carry

        lax.fori_loop(0, NRQ, initz_q, 0)

        for z0_hbm, zout_hbm in ((z0a_hbm, zouta_hbm), (z0b_hbm, zoutb_hbm)):
            def init_q(q, carry, z0_hbm=z0_hbm, zout_hbm=zout_hbm):
                rq = c * NP + r0 + q * SUB
                pltpu.sync_copy(z0_hbm.at[pl.ds(rq, SUB)], sbuf)
                pltpu.sync_copy(sbuf, zout_hbm.at[pl.ds(rq, SUB)])
                return carry

            lax.fori_loop(0, NRQ, init_q, 0)
        plsc.subcore_barrier()

        def gather(i, slot, zout_hbm):
            return pltpu.make_async_copy(
                zout_hbm.at[row_v.at[i]], stag.at[slot], semg[slot])

        def scat(i, slot):
            return pltpu.make_async_copy(
                stag.at[slot], s_sh.at[col_v.at[i]], semsc[slot])

        def hop(h, carry):
            for zout_hbm in (zouta_hbm, zoutb_hbm):
                for k in range(3):
                    gather(k, k, zout_hbm).start()

                def chunk_grp(i6, carry2, zout_hbm=zout_hbm):
                    i_base = i6 * NBUF
                    for k in range(NBUF):
                        i = i_base + k
                        ks = (k + 3) % NBUF
                        if k < 3:
                            @pl.when(i6 > 0)
                            def _(i=i, ks=ks):
                                scat(i - 3, ks).wait()
                            gather(i + 3, ks, zout_hbm).start()
                        else:
                            scat(i - 3, ks).wait()

                            @pl.when(i6 < nits - 1)
                            def _(i=i, ks=ks):
                                gather(i + 3, ks, zout_hbm).start()
                        gather(i, k, zout_hbm).wait()
                        scat(i, k).start(add=True)
                    return carry2

                lax.fori_loop(0, nits, chunk_grp, 0)
                for i in (nchunk - 3, nchunk - 2, nchunk - 1):
                    scat(i, i % NBUF).wait()
                plsc.subcore_barrier()

                def scale_q(q, carry2, zout_hbm=zout_hbm):
                    rq = r0 + q * SUB
                    pltpu.sync_copy(s_sh.at[pl.ds(rq, SUB)], sbuf)
                    pltpu.sync_copy(d2e_hbm.at[pl.ds(rq, SUB)], d2b)

                    @plsc.parallel_loop(0, SUB * (FC // 16), unroll=8)
                    def _(u):
                        r = u // (FC // 16)
                        cc = (u % (FC // 16)) * 16
                        sbuf[r, pl.ds(cc, 16)] = (
                            sbuf[r, pl.ds(cc, 16)] * d2b[r, pl.ds(cc, 16)])

                    pltpu.sync_copy(sbuf,
                                    zout_hbm.at[pl.ds(c * NP + rq, SUB)])
                    zero_sbuf()
                    pltpu.sync_copy(sbuf, s_sh.at[pl.ds(rq, SUB)])
                    return carry2

                lax.fori_loop(0, NRQ, scale_q, 0)
                plsc.subcore_barrier()
            return carry

        lax.fori_loop(0, K_HOPS, hop, 0)

    return hops_kernel(z0a, z0b, d2e, row_b, col_b)


def _tc_prep(degp):
    def body(dp_ref, dinv_ref, sdeg_ref, d2_ref):
        deg = dp_ref[0] + dp_ref[1]
        fid = (jax.lax.broadcasted_iota(jnp.int32, (NP // F, F), 0) * F
               + jax.lax.broadcasted_iota(jnp.int32, (NP // F, F), 1))
        mask = fid < N
        degs = jnp.maximum(deg, 1.0)
        dinv_ref[...] = jnp.where(mask, jax.lax.rsqrt(degs), 0.0)
        sdeg_ref[...] = jnp.where(mask, jnp.sqrt(degs), 0.0)
        d2_ref[...] = jnp.where(mask, 1.0 / degs, 0.0)

    shp = jax.ShapeDtypeStruct((NP // F, F), jnp.float32)
    return pl.pallas_call(body, out_shape=(shp, shp, shp))(degp)


def _tc_conv(x, w1r, w2t, w3t, b1, b2, b3):
    def body(x_ref, w1_ref, w2_ref, w3_ref, b1_ref, b2_ref, b3_ref, y_ref):
        xb = x_ref[0]
        x2 = xb.reshape(BN, T // 2, 2)
        h1 = jnp.zeros((BN, 30, F), jnp.float32)
        for tap in range(5):
            m = tap // 2
            par = tap % 2
            sl = x2[:, m:m + 30, par]
            h1 = h1 + sl[:, :, None] * w1_ref[tap][None, :, :]
        h1 = _leaky(h1 + b1_ref[0][None, None, :])

        h1r = h1.reshape(BN, 15, 2, F)
        o2 = jnp.zeros((BN * 13, F), jnp.float32)
        for tap in range(5):
            m = tap // 2
            par = tap % 2
            sl = h1r[:, m:m + 13, par, :].reshape(BN * 13, F)
            o2 = o2 + jnp.dot(sl, w2_ref[tap],
                              preferred_element_type=jnp.float32)
        h2 = _leaky(o2.reshape(BN, 13, F) + b2_ref[0][None, None, :])

        h2p = jnp.concatenate(
            [h2, jnp.zeros((BN, 1, F), jnp.float32)], axis=1)
        h2r = h2p.reshape(BN, 7, 2, F)
        o3 = jnp.zeros((BN * 5, F), jnp.float32)
        for tap in range(5):
            m = tap // 2
            par = tap % 2
            sl = h2r[:, m:m + 5, par, :].reshape(BN * 5, F)
            o3 = o3 + jnp.dot(sl, w3_ref[tap],
                              preferred_element_type=jnp.float32)
        h3 = o3.reshape(BN, 5, F) + b3_ref[0][None, None, :]
        y_ref[0] = jnp.max(h3, axis=1)

    grid = (B, NBLK)
    return pl.pallas_call(
        body,
        grid=grid,
        in_specs=[
            pl.BlockSpec((1, BN, T), lambda b, j: (b, j, 0)),
            pl.BlockSpec((5, 1, F), lambda b, j: (0, 0, 0)),
            pl.BlockSpec((5, F, F), lambda b, j: (0, 0, 0)),
            pl.BlockSpec((5, F, F), lambda b, j: (0, 0, 0)),
            pl.BlockSpec((1, F), lambda b, j: (0, 0)),
            pl.BlockSpec((1, F), lambda b, j: (0, 0)),
            pl.BlockSpec((1, F), lambda b, j: (0, 0)),
        ],
        out_specs=pl.BlockSpec((1, BN, F), lambda b, j: (b, j, 0)),
        out_shape=jax.ShapeDtypeStruct((B, N, F), jnp.float32),
    )(x, w1r, w2t, w3t, b1, b2, b3)


def _tc_scale_z0(y3, dinv3, d2_3):
    def body(y_ref, dinv_ref, d2_ref, za_ref, zb_ref, d2e_ref):
        j = pl.program_id(1)
        dv = dinv_ref[0, 0].reshape(BN, 1)
        rows = jax.lax.broadcasted_iota(jnp.int32, (BN, 1), 0) + j * BN
        z = jnp.where(rows < N, dv * y_ref[0], 0.0)
        za_ref[0] = z[:, :FC]
        zb_ref[0] = z[:, FC:]
        d2e_ref[...] = jnp.broadcast_to(d2_ref[0, 0].reshape(BN, 1), (BN, FC))

    zshape = jax.ShapeDtypeStruct((B, NP, FC), jnp.float32)
    return pl.pallas_call(
        body,
        grid=(B, NBLK),
        in_specs=[
            pl.BlockSpec((1, BN, F), lambda b, j: (b, j, 0)),
            pl.BlockSpec((1, 1, BN), lambda b, j: (j, 0, 0)),
            pl.BlockSpec((1, 1, BN), lambda b, j: (j, 0, 0)),
        ],
        out_specs=[
            pl.BlockSpec((1, BN, FC), lambda b, j: (b, j, 0)),
            pl.BlockSpec((1, BN, FC), lambda b, j: (b, j, 0)),
            pl.BlockSpec((BN, FC), lambda b, j: (j, 0)),
        ],
        out_shape=[zshape, zshape,
                   jax.ShapeDtypeStruct((NP, FC), jnp.float32)],
    )(y3, dinv3, d2_3)


def _tc_mix1(za, zb, sdeg3, dinv3, gw1t, gb1):
    def body(za_ref, zb_ref, sdeg_ref, dinv_ref, w_ref, b_ref,
             oa_ref, ob_ref):
        sv = sdeg_ref[0, 0].reshape(BN, 1)
        dv = dinv_ref[0, 0].reshape(BN, 1)
        x39 = sv * jnp.concatenate([za_ref[0], zb_ref[0]], axis=-1)
        h = _leaky(jnp.dot(x39, w_ref[...],
                           preferred_element_type=jnp.float32)
                   + b_ref[0][None, :])
        zo = dv * h
        oa_ref[0] = zo[:, :FC]
        ob_ref[0] = zo[:, FC:]

    zshape = jax.ShapeDtypeStruct((B, NP, FC), jnp.float32)
    return pl.pallas_call(
        body,
        grid=(B, NBLK),
        in_specs=[
            pl.BlockSpec((1, BN, FC), lambda b, j: (b, j, 0)),
            pl.BlockSpec((1, BN, FC), lambda b, j: (b, j, 0)),
            pl.BlockSpec((1, 1, BN), lambda b, j: (j, 0, 0)),
            pl.BlockSpec((1, 1, BN), lambda b, j: (j, 0, 0)),
            pl.BlockSpec((F, F), lambda b, j: (0, 0)),
            pl.BlockSpec((1, F), lambda b, j: (0, 0)),
        ],
        out_specs=[
            pl.BlockSpec((1, BN, FC), lambda b, j: (b, j, 0)),
            pl.BlockSpec((1, BN, FC), lambda b, j: (b, j, 0)),
        ],
        out_shape=[zshape, zshape],
    )(za, zb, sdeg3, dinv3, gw1t, gb1)


def _tc_mix2(za, zb, sdeg3, gw2t, gb2):
    def body(za_ref, zb_ref, sdeg_ref, w_ref, b_ref, o_ref):
        j = pl.program_id(1)
        sv = sdeg_ref[0, 0].reshape(BN, 1)
        x39 = sv * jnp.concatenate([za_ref[0], zb_ref[0]], axis=-1)
        h = _leaky(jnp.dot(x39, w_ref[...],
                           preferred_element_type=jnp.float32)
                   + b_ref[0][None, :])
        rows = jax.lax.broadcasted_iota(jnp.int32, (BN, 1), 0) + j * BN
        hm = jnp.where(rows < N, h, 0.0)
        ps = jnp.sum(hm.reshape(BN // 8, 8, F), axis=0)[None]

        @pl.when(j == 0)
        def _():
            o_ref[...] = ps

        @pl.when(j != 0)
        def _():
            o_ref[...] = o_ref[...] + ps

    return pl.pallas_call(
        body,
        grid=(B, NBLK),
        in_specs=[
            pl.BlockSpec((1, BN, FC), lambda b, j: (b, j, 0)),
            pl.BlockSpec((1, BN, FC), lambda b, j: (b, j, 0)),
            pl.BlockSpec((1, 1, BN), lambda b, j: (j, 0, 0)),
            pl.BlockSpec((F, F), lambda b, j: (0, 0)),
            pl.BlockSpec((1, F), lambda b, j: (0, 0)),
        ],
        out_specs=pl.BlockSpec((1, 8, F), lambda b, j: (b, 0, 0)),
        out_shape=jax.ShapeDtypeStruct((B, 8, F), jnp.float32),
    )(za, zb, sdeg3, gw2t, gb2)


def _tc_head(hsum, linwt_pad, linb_pad):
    def body(h_ref, w_ref, b_ref, o_ref):
        hs = jnp.sum(h_ref[...], axis=1)
        o_ref[...] = (jnp.dot(hs * (1.0 / N), w_ref[...],
                              preferred_element_type=jnp.float32)
                      + b_ref[...])

    return pl.pallas_call(
        body,
        out_shape=jax.ShapeDtypeStruct((B, F), jnp.float32),
    )(hsum, linwt_pad, linb_pad)


def kernel(x, edge_index, W1, b1, W2, b2, W3, b3, gW1, gb1, gW2, gb2,
           linW, linb):
    E = edge_index.shape[1]
    e_real = E + N
    epad = ((e_real + 32 * EC - 1) // (32 * EC)) * (32 * EC)
    npad_e = epad - e_real

    loop = jnp.arange(N, dtype=jnp.int32)
    pad_idx = N + (jnp.arange(npad_e, dtype=jnp.int32) % (NP - N))
    row_full = jnp.concatenate([edge_index[0], loop, pad_idx])
    col_full = jnp.concatenate([edge_index[1], loop, pad_idx])

    col_a = col_full.reshape(32, epad // (32 * EC), EC)
    row_1 = row_full.reshape(NTEC, epad // (NTEC * EC), EC)
    col_b = col_full.reshape(NTEC, epad // (NTEC * EC), EC)
    row_b = jnp.stack([row_1, row_1 + NP])

    ones_hbm = jnp.ones((EC,), jnp.float32)
    zrow_hbm = jnp.zeros((RPT,), jnp.float32)

    w1r = jnp.transpose(W1, (2, 1, 0))
    w2t = jnp.transpose(W2, (2, 1, 0))
    w3t = jnp.transpose(W3, (2, 1, 0))
    b1r = b1.reshape(1, F)
    b2r = b2.reshape(1, F)
    b3r = b3.reshape(1, F)
    gw1t = jnp.transpose(gW1)
    gw2t = jnp.transpose(gW2)
    linwt_pad = jnp.zeros((F, F), jnp.float32).at[:, :2].set(jnp.transpose(linW))
    linb_pad = jnp.zeros((B, F), jnp.float32).at[:, :2].set(linb[None, :])

    degp = _sc_deg(col_a, ones_hbm, zrow_hbm)
    degp3 = degp.reshape(B, NP // F, F)

    dinv2, sdeg2, d2_2 = _tc_prep(degp3)
    dinv3 = dinv2.reshape(NBLK, 1, BN)
    sdeg3 = sdeg2.reshape(NBLK, 1, BN)
    d2_3 = d2_2.reshape(NBLK, 1, BN)

    y3 = _tc_conv(x, w1r, w2t, w3t, b1r, b2r, b3r)

    z0a, z0b_half, d2e = _tc_scale_z0(y3, dinv3, d2_3)

    za1, zb1 = _sc_hops(z0a.reshape(B * NP, FC), z0b_half.reshape(B * NP, FC),
                        d2e, row_b, col_b)
    ma, mb = _tc_mix1(za1.reshape(B, NP, FC), zb1.reshape(B, NP, FC),
                      sdeg3, dinv3, gw1t, gb1.reshape(1, F))
    za2, zb2 = _sc_hops(ma.reshape(B * NP, FC), mb.reshape(B * NP, FC),
                        d2e, row_b, col_b)
    hsum = _tc_mix2(za2.reshape(B, NP, FC), zb2.reshape(B, NP, FC),
                    sdeg3, gw2t, gb2.reshape(1, F))
    outp = _tc_head(hsum, linwt_pad, linb_pad)

    out = outp[:, :2]
    y = y3.reshape(B * N, F)
    return (out, y)


# speedup vs baseline: 12.8149x; 1.1175x over previous
import functools

import jax
import jax.numpy as jnp
from jax import lax
from jax.experimental import pallas as pl
from jax.experimental.pallas import tpu as pltpu
from jax.experimental.pallas import tpu_sc as plsc

B = 2
N = 10000
T = 64
F = 128
NTEC = 16
NCORE = 2
RPT = 640
NP = NTEC * RPT
FC = 64
SUB = 32
NRQ = RPT // SUB
EC = 128
NBUF = 6
K_HOPS = 39
BN = 256
NBLK = 40
NEG_SLOPE = 0.01


def _leaky(v):
    return jnp.where(v > 0, v, NEG_SLOPE * v)


def _sc_deg(col_a, ones_hbm, zrow_hbm):
    nchunk = col_a.shape[1]
    mesh = plsc.VectorSubcoreMesh(core_axis_name="c", subcore_axis_name="s")

    @functools.partial(
        pl.kernel,
        mesh=mesh,
        out_type=jax.ShapeDtypeStruct((NCORE * NP,), jnp.float32),
        scratch_types=[
            pltpu.VMEM((nchunk, EC), jnp.int32),
            pltpu.VMEM((EC,), jnp.float32),
            pltpu.VMEM((RPT,), jnp.float32),
            pltpu.VMEM_SHARED((NP,), jnp.float32),
        ],
    )
    def deg_kernel(col_hbm, ones_in, zrow_in, deg_out, col_v, ones_v, buf_v,
                   deg_sh):
        c = lax.axis_index("c")
        s = lax.axis_index("s")
        w = s * NCORE + c
        pltpu.sync_copy(col_hbm.at[w], col_v)
        pltpu.sync_copy(ones_in, ones_v)
        pltpu.sync_copy(zrow_in, buf_v)
        pltpu.sync_copy(buf_v, deg_sh.at[pl.ds(s * RPT, RPT)])
        plsc.subcore_barrier()

        def body(i, carry):
            pltpu.sync_copy(ones_v, deg_sh.at[col_v.at[i]], add=True)
            return carry

        lax.fori_loop(0, nchunk, body, 0)
        plsc.subcore_barrier()
        pltpu.sync_copy(deg_sh.at[pl.ds(s * RPT, RPT)], buf_v)
        pltpu.sync_copy(buf_v, deg_out.at[pl.ds(c * NP + s * RPT, RPT)])

    return deg_kernel(col_a, ones_hbm, zrow_hbm)


def _sc_hops(z0a, z0b, d2e, row_b, col_b):
    nchunk = col_b.shape[1]
    nits = nchunk // NBUF
    mesh = plsc.VectorSubcoreMesh(core_axis_name="c", subcore_axis_name="s")
    zshape = jax.ShapeDtypeStruct((B * NP, FC), jnp.float32)

    @functools.partial(
        pl.kernel,
        mesh=mesh,
        out_type=(zshape, zshape),
        scratch_types=[
            pltpu.VMEM((nchunk, EC), jnp.int32),
            pltpu.VMEM((nchunk, EC), jnp.int32),
            pltpu.VMEM((NBUF, EC, FC), jnp.float32),
            pltpu.VMEM((4, SUB, FC), jnp.float32),
            pltpu.VMEM((2, SUB, FC), jnp.float32),
            pltpu.VMEM((SUB, FC), jnp.float32),
            pltpu.VMEM_SHARED((NP, FC), jnp.float32),
        ] + [pltpu.SemaphoreType.DMA] * (2 * NBUF + 11),
        compiler_params=pltpu.CompilerParams(use_tc_tiling_on_sc=False),
    )
    def hops_kernel(z0a_hbm, z0b_hbm, d2e_hbm, row_hbm, col_hbm,
                    zouta_hbm, zoutb_hbm,
                    row_v, col_v, stag, sbuf, d2b, zbuf, s_sh, *sems):
        semg = sems[:NBUF]
        semsc = sems[NBUF:2 * NBUF]
        semr = sems[2 * NBUF:2 * NBUF + 4]
        semw = sems[2 * NBUF + 4:2 * NBUF + 8]
        semd = sems[2 * NBUF + 8:2 * NBUF + 10]
        semz = sems[2 * NBUF + 10]
        c = lax.axis_index("c")
        s = lax.axis_index("s")
        r0 = s * RPT
        pltpu.sync_copy(row_hbm.at[c, s], row_v)
        pltpu.sync_copy(col_hbm.at[s], col_v)

        @plsc.parallel_loop(0, SUB * (FC // 16), unroll=8)
        def _(u):
            zbuf[u // (FC // 16), pl.ds((u % (FC // 16)) * 16, 16)] = (
                jnp.zeros((16,), jnp.float32))

        def initz_q(q, carry):
            pltpu.sync_copy(zbuf, s_sh.at[pl.ds(r0 + q * SUB, SUB)])
            return carry

        lax.fori_loop(0, NRQ, initz_q, 0)

        for z0_hbm, zout_hbm in ((z0a_hbm, zouta_hbm), (z0b_hbm, zoutb_hbm)):
            def init_q(q, carry, z0_hbm=z0_hbm, zout_hbm=zout_hbm):
                rq = c * NP + r0 + q * SUB
                pltpu.sync_copy(z0_hbm.at[pl.ds(rq, SUB)], sbuf.at[0])
                pltpu.sync_copy(sbuf.at[0], zout_hbm.at[pl.ds(rq, SUB)])
                return carry

            lax.fori_loop(0, NRQ, init_q, 0)
        plsc.subcore_barrier()

        def gather(i, slot, zout_hbm):
            return pltpu.make_async_copy(
                zout_hbm.at[row_v.at[i]], stag.at[slot], semg[slot])

        def scat(i, slot):
            return pltpu.make_async_copy(
                stag.at[slot], s_sh.at[col_v.at[i]], semsc[slot])

        def hop(h, carry):
            for zout_hbm in (zouta_hbm, zoutb_hbm):
                for k in range(3):
                    gather(k, k, zout_hbm).start()

                def chunk_grp(i6, carry2, zout_hbm=zout_hbm):
                    i_base = i6 * NBUF
                    for k in range(NBUF):
                        i = i_base + k
                        ks = (k + 3) % NBUF
                        if k < 3:
                            @pl.when(i6 > 0)
                            def _(i=i, ks=ks):
                                scat(i - 3, ks).wait()
                            gather(i + 3, ks, zout_hbm).start()
                        else:
                            scat(i - 3, ks).wait()

                            @pl.when(i6 < nits - 1)
                            def _(i=i, ks=ks):
                                gather(i + 3, ks, zout_hbm).start()
                        gather(i, k, zout_hbm).wait()
                        scat(i, k).start(add=True)
                    return carry2

                lax.fori_loop(0, nits, chunk_grp, 0)
                for i in (nchunk - 3, nchunk - 2, nchunk - 1):
                    scat(i, i % NBUF).wait()
                plsc.subcore_barrier()

                def rd_s(q, b):
                    return pltpu.make_async_copy(
                        s_sh.at[pl.ds(r0 + q * SUB, SUB)], sbuf.at[b],
                        semr[b])

                def rd_d2(q, b2):
                    return pltpu.make_async_copy(
                        d2e_hbm.at[pl.ds(r0 + q * SUB, SUB)], d2b.at[b2],
                        semd[b2])

                def wr_z(q, b, zout_hbm=zout_hbm):
                    return pltpu.make_async_copy(
                        sbuf.at[b],
                        zout_hbm.at[pl.ds(c * NP + r0 + q * SUB, SUB)],
                        semw[b])

                def wr_zero(q):
                    return pltpu.make_async_copy(
                        zbuf, s_sh.at[pl.ds(r0 + q * SUB, SUB)], semz)

                rd_s(0, 0).start()
                rd_d2(0, 0).start()
                rd_s(1, 1).start()
                rd_d2(1, 1).start()

                def scale_grp(q4, carry2, zout_hbm=zout_hbm):
                    for b in range(4):
                        q = q4 * 4 + b
                        b2 = b % 2
                        rd_s(q, b).wait()
                        rd_d2(q, b2).wait()

                        @plsc.parallel_loop(0, SUB * (FC // 16), unroll=8)
                        def _(u, b=b, b2=b2):
                            r = u // (FC // 16)
                            cc = (u % (FC // 16)) * 16
                            sbuf[b, r, pl.ds(cc, 16)] = (
                                sbuf[b, r, pl.ds(cc, 16)]
                                * d2b[b2, r, pl.ds(cc, 16)])

                        wr_z(q, b).start()
                        wr_zero(q).start()

                        @pl.when(q + 2 < NRQ)
                        def _(q=q, b=b, b2=b2):
                            @pl.when(q >= 2)
                            def _():
                                wr_z(q - 2, (b + 2) % 4).wait()
                            rd_s(q + 2, (b + 2) % 4).start()
                            rd_d2(q + 2, b2).start()
                    return carry2

                lax.fori_loop(0, NRQ // 4, scale_grp, 0)
                for q in (NRQ - 4, NRQ - 3, NRQ - 2, NRQ - 1):
                    wr_z(q, q % 4).wait()

                def drain_z(q, carry2):
                    wr_zero(q).wait()
                    return carry2

                lax.fori_loop(0, NRQ, drain_z, 0)
                plsc.subcore_barrier()
            return carry

        lax.fori_loop(0, K_HOPS, hop, 0)

    return hops_kernel(z0a, z0b, d2e, row_b, col_b)


def _tc_prep(degp):
    def body(dp_ref, dinv_ref, sdeg_ref, d2_ref):
        deg = dp_ref[0] + dp_ref[1]
        fid = (jax.lax.broadcasted_iota(jnp.int32, (NP // F, F), 0) * F
               + jax.lax.broadcasted_iota(jnp.int32, (NP // F, F), 1))
        mask = fid < N
        degs = jnp.maximum(deg, 1.0)
        dinv_ref[...] = jnp.where(mask, jax.lax.rsqrt(degs), 0.0)
        sdeg_ref[...] = jnp.where(mask, jnp.sqrt(degs), 0.0)
        d2_ref[...] = jnp.where(mask, 1.0 / degs, 0.0)

    shp = jax.ShapeDtypeStruct((NP // F, F), jnp.float32)
    return pl.pallas_call(body, out_shape=(shp, shp, shp))(degp)


def _tc_conv(x, w1r, w2t, w3t, b1, b2, b3):
    def body(x_ref, w1_ref, w2_ref, w3_ref, b1_ref, b2_ref, b3_ref, y_ref):
        xb = x_ref[0]
        x2 = xb.reshape(BN, T // 2, 2)
        h1 = jnp.zeros((BN, 30, F), jnp.float32)
        for tap in range(5):
            m = tap // 2
            par = tap % 2
            sl = x2[:, m:m + 30, par]
            h1 = h1 + sl[:, :, None] * w1_ref[tap][None, :, :]
        h1 = _leaky(h1 + b1_ref[0][None, None, :])

        h1r = h1.reshape(BN, 15, 2, F)
        o2 = jnp.zeros((BN * 13, F), jnp.float32)
        for tap in range(5):
            m = tap // 2
            par = tap % 2
            sl = h1r[:, m:m + 13, par, :].reshape(BN * 13, F)
            o2 = o2 + jnp.dot(sl, w2_ref[tap],
                              preferred_element_type=jnp.float32)
        h2 = _leaky(o2.reshape(BN, 13, F) + b2_ref[0][None, None, :])

        h2p = jnp.concatenate(
            [h2, jnp.zeros((BN, 1, F), jnp.float32)], axis=1)
        h2r = h2p.reshape(BN, 7, 2, F)
        o3 = jnp.zeros((BN * 5, F), jnp.float32)
        for tap in range(5):
            m = tap // 2
            par = tap % 2
            sl = h2r[:, m:m + 5, par, :].reshape(BN * 5, F)
            o3 = o3 + jnp.dot(sl, w3_ref[tap],
                              preferred_element_type=jnp.float32)
        h3 = o3.reshape(BN, 5, F) + b3_ref[0][None, None, :]
        y_ref[0] = jnp.max(h3, axis=1)

    grid = (B, NBLK)
    return pl.pallas_call(
        body,
        grid=grid,
        in_specs=[
            pl.BlockSpec((1, BN, T), lambda b, j: (b, j, 0)),
            pl.BlockSpec((5, 1, F), lambda b, j: (0, 0, 0)),
            pl.BlockSpec((5, F, F), lambda b, j: (0, 0, 0)),
            pl.BlockSpec((5, F, F), lambda b, j: (0, 0, 0)),
            pl.BlockSpec((1, F), lambda b, j: (0, 0)),
            pl.BlockSpec((1, F), lambda b, j: (0, 0)),
            pl.BlockSpec((1, F), lambda b, j: (0, 0)),
        ],
        out_specs=pl.BlockSpec((1, BN, F), lambda b, j: (b, j, 0)),
        out_shape=jax.ShapeDtypeStruct((B, N, F), jnp.float32),
    )(x, w1r, w2t, w3t, b1, b2, b3)


def _tc_scale_z0(y3, dinv3, d2_3):
    def body(y_ref, dinv_ref, d2_ref, za_ref, zb_ref, d2e_ref):
        j = pl.program_id(1)
        dv = dinv_ref[0, 0].reshape(BN, 1)
        rows = jax.lax.broadcasted_iota(jnp.int32, (BN, 1), 0) + j * BN
        z = jnp.where(rows < N, dv * y_ref[0], 0.0)
        za_ref[0] = z[:, :FC]
        zb_ref[0] = z[:, FC:]
        d2e_ref[...] = jnp.broadcast_to(d2_ref[0, 0].reshape(BN, 1), (BN, FC))

    zshape = jax.ShapeDtypeStruct((B, NP, FC), jnp.float32)
    return pl.pallas_call(
        body,
        grid=(B, NBLK),
        in_specs=[
            pl.BlockSpec((1, BN, F), lambda b, j: (b, j, 0)),
            pl.BlockSpec((1, 1, BN), lambda b, j: (j, 0, 0)),
            pl.BlockSpec((1, 1, BN), lambda b, j: (j, 0, 0)),
        ],
        out_specs=[
            pl.BlockSpec((1, BN, FC), lambda b, j: (b, j, 0)),
            pl.BlockSpec((1, BN, FC), lambda b, j: (b, j, 0)),
            pl.BlockSpec((BN, FC), lambda b, j: (j, 0)),
        ],
        out_shape=[zshape, zshape,
                   jax.ShapeDtypeStruct((NP, FC), jnp.float32)],
    )(y3, dinv3, d2_3)


def _tc_mix1(za, zb, sdeg3, dinv3, gw1t, gb1):
    def body(za_ref, zb_ref, sdeg_ref, dinv_ref, w_ref, b_ref,
             oa_ref, ob_ref):
        sv = sdeg_ref[0, 0].reshape(BN, 1)
        dv = dinv_ref[0, 0].reshape(BN, 1)
        x39 = sv * jnp.concatenate([za_ref[0], zb_ref[0]], axis=-1)
        h = _leaky(jnp.dot(x39, w_ref[...],
                           preferred_element_type=jnp.float32)
                   + b_ref[0][None, :])
        zo = dv * h
        oa_ref[0] = zo[:, :FC]
        ob_ref[0] = zo[:, FC:]

    zshape = jax.ShapeDtypeStruct((B, NP, FC), jnp.float32)
    return pl.pallas_call(
        body,
        grid=(B, NBLK),
        in_specs=[
            pl.BlockSpec((1, BN, FC), lambda b, j: (b, j, 0)),
            pl.BlockSpec((1, BN, FC), lambda b, j: (b, j, 0)),
            pl.BlockSpec((1, 1, BN), lambda b, j: (j, 0, 0)),
            pl.BlockSpec((1, 1, BN), lambda b, j: (j, 0, 0)),
            pl.BlockSpec((F, F), lambda b, j: (0, 0)),
            pl.BlockSpec((1, F), lambda b, j: (0, 0)),
        ],
        out_specs=[
            pl.BlockSpec((1, BN, FC), lambda b, j: (b, j, 0)),
            pl.BlockSpec((1, BN, FC), lambda b, j: (b, j, 0)),
        ],
        out_shape=[zshape, zshape],
    )(za, zb, sdeg3, dinv3, gw1t, gb1)


def _tc_mix2(za, zb, sdeg3, gw2t, gb2):
    def body(za_ref, zb_ref, sdeg_ref, w_ref, b_ref, o_ref):
        j = pl.program_id(1)
        sv = sdeg_ref[0, 0].reshape(BN, 1)
        x39 = sv * jnp.concatenate([za_ref[0], zb_ref[0]], axis=-1)
        h = _leaky(jnp.dot(x39, w_ref[...],
                           preferred_element_type=jnp.float32)
                   + b_ref[0][None, :])
        rows = jax.lax.broadcasted_iota(jnp.int32, (BN, 1), 0) + j * BN
        hm = jnp.where(rows < N, h, 0.0)
        ps = jnp.sum(hm.reshape(BN // 8, 8, F), axis=0)[None]

        @pl.when(j == 0)
        def _():
            o_ref[...] = ps

        @pl.when(j != 0)
        def _():
            o_ref[...] = o_ref[...] + ps

    return pl.pallas_call(
        body,
        grid=(B, NBLK),
        in_specs=[
            pl.BlockSpec((1, BN, FC), lambda b, j: (b, j, 0)),
            pl.BlockSpec((1, BN, FC), lambda b, j: (b, j, 0)),
            pl.BlockSpec((1, 1, BN), lambda b, j: (j, 0, 0)),
            pl.BlockSpec((F, F), lambda b, j: (0, 0)),
            pl.BlockSpec((1, F), lambda b, j: (0, 0)),
        ],
        out_specs=pl.BlockSpec((1, 8, F), lambda b, j: (b, 0, 0)),
        out_shape=jax.ShapeDtypeStruct((B, 8, F), jnp.float32),
    )(za, zb, sdeg3, gw2t, gb2)


def _tc_head(hsum, linwt_pad, linb_pad):
    def body(h_ref, w_ref, b_ref, o_ref):
        hs = jnp.sum(h_ref[...], axis=1)
        o_ref[...] = (jnp.dot(hs * (1.0 / N), w_ref[...],
                              preferred_element_type=jnp.float32)
                      + b_ref[...])

    return pl.pallas_call(
        body,
        out_shape=jax.ShapeDtypeStruct((B, F), jnp.float32),
    )(hsum, linwt_pad, linb_pad)


def kernel(x, edge_index, W1, b1, W2, b2, W3, b3, gW1, gb1, gW2, gb2,
           linW, linb):
    E = edge_index.shape[1]
    e_real = E + N
    epad = ((e_real + 32 * EC - 1) // (32 * EC)) * (32 * EC)
    npad_e = epad - e_real

    loop = jnp.arange(N, dtype=jnp.int32)
    pad_idx = N + (jnp.arange(npad_e, dtype=jnp.int32) % (NP - N))
    row_full = jnp.concatenate([edge_index[0], loop, pad_idx])
    col_full = jnp.concatenate([edge_index[1], loop, pad_idx])

    col_a = col_full.reshape(32, epad // (32 * EC), EC)
    row_1 = row_full.reshape(NTEC, epad // (NTEC * EC), EC)
    col_b = col_full.reshape(NTEC, epad // (NTEC * EC), EC)
    row_b = jnp.stack([row_1, row_1 + NP])

    ones_hbm = jnp.ones((EC,), jnp.float32)
    zrow_hbm = jnp.zeros((RPT,), jnp.float32)

    w1r = jnp.transpose(W1, (2, 1, 0))
    w2t = jnp.transpose(W2, (2, 1, 0))
    w3t = jnp.transpose(W3, (2, 1, 0))
    b1r = b1.reshape(1, F)
    b2r = b2.reshape(1, F)
    b3r = b3.reshape(1, F)
    gw1t = jnp.transpose(gW1)
    gw2t = jnp.transpose(gW2)
    linwt_pad = jnp.zeros((F, F), jnp.float32).at[:, :2].set(jnp.transpose(linW))
    linb_pad = jnp.zeros((B, F), jnp.float32).at[:, :2].set(linb[None, :])

    degp = _sc_deg(col_a, ones_hbm, zrow_hbm)
    degp3 = degp.reshape(B, NP // F, F)

    dinv2, sdeg2, d2_2 = _tc_prep(degp3)
    dinv3 = dinv2.reshape(NBLK, 1, BN)
    sdeg3 = sdeg2.reshape(NBLK, 1, BN)
    d2_3 = d2_2.reshape(NBLK, 1, BN)

    y3 = _tc_conv(x, w1r, w2t, w3t, b1r, b2r, b3r)

    z0a, z0b_half, d2e = _tc_scale_z0(y3, dinv3, d2_3)

    za1, zb1 = _sc_hops(z0a.reshape(B * NP, FC), z0b_half.reshape(B * NP, FC),
                        d2e, row_b, col_b)
    ma, mb = _tc_mix1(za1.reshape(B, NP, FC), zb1.reshape(B, NP, FC),
                      sdeg3, dinv3, gw1t, gb1.reshape(1, F))
    za2, zb2 = _sc_hops(ma.reshape(B * NP, FC), mb.reshape(B * NP, FC),
                        d2e, row_b, col_b)
    hsum = _tc_mix2(za2.reshape(B, NP, FC), zb2.reshape(B, NP, FC),
                    sdeg3, gw2t, gb2.reshape(1, F))
    outp = _tc_head(hsum, linwt_pad, linb_pad)

    out = outp[:, :2]
    y = y3.reshape(B * N, F)
    return (out, y)


# speedup vs baseline: 12.8198x; 1.0004x over previous
import functools

import jax
import jax.numpy as jnp
from jax import lax
from jax.experimental import pallas as pl
from jax.experimental.pallas import tpu as pltpu
from jax.experimental.pallas import tpu_sc as plsc

B = 2
N = 10000
T = 64
F = 128
NTEC = 16
NCORE = 2
RPT = 640
NP = NTEC * RPT
FC = 64
SUB = 32
NRQ = RPT // SUB
EC = 128
NBUF = 6
K_HOPS = 39
BN = 256
NBLK = 40
NEG_SLOPE = 0.01


def _leaky(v):
    return jnp.where(v > 0, v, NEG_SLOPE * v)


def _sc_deg(col_a, ones_hbm, zrow_hbm):
    nchunk = col_a.shape[1]
    mesh = plsc.VectorSubcoreMesh(core_axis_name="c", subcore_axis_name="s")

    @functools.partial(
        pl.kernel,
        mesh=mesh,
        out_type=jax.ShapeDtypeStruct((NCORE * NP,), jnp.float32),
        scratch_types=[
            pltpu.VMEM((nchunk, EC), jnp.int32),
            pltpu.VMEM((EC,), jnp.float32),
            pltpu.VMEM((RPT,), jnp.float32),
            pltpu.VMEM_SHARED((NP,), jnp.float32),
        ],
    )
    def deg_kernel(col_hbm, ones_in, zrow_in, deg_out, col_v, ones_v, buf_v,
                   deg_sh):
        c = lax.axis_index("c")
        s = lax.axis_index("s")
        w = s * NCORE + c
        pltpu.sync_copy(col_hbm.at[w], col_v)
        pltpu.sync_copy(ones_in, ones_v)
        pltpu.sync_copy(zrow_in, buf_v)
        pltpu.sync_copy(buf_v, deg_sh.at[pl.ds(s * RPT, RPT)])
        plsc.subcore_barrier()

        def body(i, carry):
            pltpu.sync_copy(ones_v, deg_sh.at[col_v.at[i]], add=True)
            return carry

        lax.fori_loop(0, nchunk, body, 0)
        plsc.subcore_barrier()
        pltpu.sync_copy(deg_sh.at[pl.ds(s * RPT, RPT)], buf_v)
        pltpu.sync_copy(buf_v, deg_out.at[pl.ds(c * NP + s * RPT, RPT)])

    return deg_kernel(col_a, ones_hbm, zrow_hbm)


def _sc_hops(z0a, z0b, d2e, row_b, col_b):
    nchunk = col_b.shape[1]
    nits = nchunk // NBUF
    mesh = plsc.VectorSubcoreMesh(core_axis_name="c", subcore_axis_name="s")
    zshape = jax.ShapeDtypeStruct((B * NP, FC), jnp.float32)

    @functools.partial(
        pl.kernel,
        mesh=mesh,
        out_type=(zshape, zshape),
        scratch_types=[
            pltpu.VMEM((nchunk, EC), jnp.int32),
            pltpu.VMEM((nchunk, EC), jnp.int32),
            pltpu.VMEM((NBUF, EC, FC), jnp.float32),
            pltpu.VMEM((4, SUB, FC), jnp.float32),
            pltpu.VMEM((2, SUB, FC), jnp.float32),
            pltpu.VMEM((SUB, FC), jnp.float32),
            pltpu.VMEM_SHARED((NP, FC), jnp.float32),
        ] + [pltpu.SemaphoreType.DMA] * (2 * NBUF + 11),
        compiler_params=pltpu.CompilerParams(use_tc_tiling_on_sc=False),
    )
    def hops_kernel(z0a_hbm, z0b_hbm, d2e_hbm, row_hbm, col_hbm,
                    zouta_hbm, zoutb_hbm,
                    row_v, col_v, stag, sbuf, d2b, zbuf, s_sh, *sems):
        semg = sems[:NBUF]
        semsc = sems[NBUF:2 * NBUF]
        semr = sems[2 * NBUF:2 * NBUF + 4]
        semw = sems[2 * NBUF + 4:2 * NBUF + 8]
        semd = sems[2 * NBUF + 8:2 * NBUF + 10]
        semz = sems[2 * NBUF + 10]
        c = lax.axis_index("c")
        s = lax.axis_index("s")
        r0 = s * RPT
        pltpu.sync_copy(row_hbm.at[c, s], row_v)
        pltpu.sync_copy(col_hbm.at[s], col_v)

        @plsc.parallel_loop(0, SUB * (FC // 16), unroll=8)
        def _(u):
            zbuf[u // (FC // 16), pl.ds((u % (FC // 16)) * 16, 16)] = (
                jnp.zeros((16,), jnp.float32))

        def initz_q(q, carry):
            pltpu.sync_copy(zbuf, s_sh.at[pl.ds(r0 + q * SUB, SUB)])
            return carry

        lax.fori_loop(0, NRQ, initz_q, 0)

        for z0_hbm, zout_hbm in ((z0a_hbm, zouta_hbm), (z0b_hbm, zoutb_hbm)):
            def init_q(q, carry, z0_hbm=z0_hbm, zout_hbm=zout_hbm):
                rq = c * NP + r0 + q * SUB
                pltpu.sync_copy(z0_hbm.at[pl.ds(rq, SUB)], sbuf.at[0])
                pltpu.sync_copy(sbuf.at[0], zout_hbm.at[pl.ds(rq, SUB)])
                return carry

            lax.fori_loop(0, NRQ, init_q, 0)
        plsc.subcore_barrier()

        def gather(i, slot, zout_hbm):
            return pltpu.make_async_copy(
                zout_hbm.at[row_v.at[i]], stag.at[slot], semg[slot])

        def scat(i, slot):
            return pltpu.make_async_copy(
                stag.at[slot], s_sh.at[col_v.at[i]], semsc[slot])

        def hop(h, carry):
            for zout_hbm in (zouta_hbm, zoutb_hbm):
                for k in range(4):
                    gather(k, k, zout_hbm).start()

                def chunk_grp(i6, carry2, zout_hbm=zout_hbm):
                    i_base = i6 * NBUF
                    for k in range(NBUF):
                        i = i_base + k
                        ks = (k + 4) % NBUF
                        if k < 2:
                            @pl.when(i6 > 0)
                            def _(i=i, ks=ks):
                                scat(i - 2, ks).wait()
                            gather(i + 4, ks, zout_hbm).start()
                        else:
                            scat(i - 2, ks).wait()

                            @pl.when(i6 < nits - 1)
                            def _(i=i, ks=ks):
                                gather(i + 4, ks, zout_hbm).start()
                        gather(i, k, zout_hbm).wait()
                        scat(i, k).start(add=True)
                    return carry2

                lax.fori_loop(0, nits, chunk_grp, 0)
                for i in (nchunk - 2, nchunk - 1):
                    scat(i, i % NBUF).wait()
                plsc.subcore_barrier()

                def rd_s(q, b):
                    return pltpu.make_async_copy(
                        s_sh.at[pl.ds(r0 + q * SUB, SUB)], sbuf.at[b],
                        semr[b])

                def rd_d2(q, b2):
                    return pltpu.make_async_copy(
                        d2e_hbm.at[pl.ds(r0 + q * SUB, SUB)], d2b.at[b2],
                        semd[b2])

                def wr_z(q, b, zout_hbm=zout_hbm):
                    return pltpu.make_async_copy(
                        sbuf.at[b],
                        zout_hbm.at[pl.ds(c * NP + r0 + q * SUB, SUB)],
                        semw[b])

                def wr_zero(q):
                    return pltpu.make_async_copy(
                        zbuf, s_sh.at[pl.ds(r0 + q * SUB, SUB)], semz)

                rd_s(0, 0).start()
                rd_d2(0, 0).start()
                rd_s(1, 1).start()
                rd_d2(1, 1).start()

                def scale_grp(q4, carry2, zout_hbm=zout_hbm):
                    for b in range(4):
                        q = q4 * 4 + b
                        b2 = b % 2
                        rd_s(q, b).wait()
                        rd_d2(q, b2).wait()

                        @plsc.parallel_loop(0, SUB * (FC // 16), unroll=8)
                        def _(u, b=b, b2=b2):
                            r = u // (FC // 16)
                            cc = (u % (FC // 16)) * 16
                            sbuf[b, r, pl.ds(cc, 16)] = (
                                sbuf[b, r, pl.ds(cc, 16)]
                                * d2b[b2, r, pl.ds(cc, 16)])

                        wr_z(q, b).start()
                        wr_zero(q).start()

                        @pl.when(q + 2 < NRQ)
                        def _(q=q, b=b, b2=b2):
                            @pl.when(q >= 2)
                            def _():
                                wr_z(q - 2, (b + 2) % 4).wait()
                            rd_s(q + 2, (b + 2) % 4).start()
                            rd_d2(q + 2, b2).start()
                    return carry2

                lax.fori_loop(0, NRQ // 4, scale_grp, 0)
                for q in (NRQ - 4, NRQ - 3, NRQ - 2, NRQ - 1):
                    wr_z(q, q % 4).wait()

                def drain_z(q, carry2):
                    wr_zero(q).wait()
                    return carry2

                lax.fori_loop(0, NRQ, drain_z, 0)
                plsc.subcore_barrier()
            return carry

        lax.fori_loop(0, K_HOPS, hop, 0)

    return hops_kernel(z0a, z0b, d2e, row_b, col_b)


def _tc_prep(degp):
    def body(dp_ref, dinv_ref, sdeg_ref, d2_ref):
        deg = dp_ref[0] + dp_ref[1]
        fid = (jax.lax.broadcasted_iota(jnp.int32, (NP // F, F), 0) * F
               + jax.lax.broadcasted_iota(jnp.int32, (NP // F, F), 1))
        mask = fid < N
        degs = jnp.maximum(deg, 1.0)
        dinv_ref[...] = jnp.where(mask, jax.lax.rsqrt(degs), 0.0)
        sdeg_ref[...] = jnp.where(mask, jnp.sqrt(degs), 0.0)
        d2_ref[...] = jnp.where(mask, 1.0 / degs, 0.0)

    shp = jax.ShapeDtypeStruct((NP // F, F), jnp.float32)
    return pl.pallas_call(body, out_shape=(shp, shp, shp))(degp)


def _tc_conv(x, w1r, w2t, w3t, b1, b2, b3):
    def body(x_ref, w1_ref, w2_ref, w3_ref, b1_ref, b2_ref, b3_ref, y_ref):
        xb = x_ref[0]
        x2 = xb.reshape(BN, T // 2, 2)
        h1 = jnp.zeros((BN, 30, F), jnp.float32)
        for tap in range(5):
            m = tap // 2
            par = tap % 2
            sl = x2[:, m:m + 30, par]
            h1 = h1 + sl[:, :, None] * w1_ref[tap][None, :, :]
        h1 = _leaky(h1 + b1_ref[0][None, None, :])

        h1r = h1.reshape(BN, 15, 2, F)
        o2 = jnp.zeros((BN * 13, F), jnp.float32)
        for tap in range(5):
            m = tap // 2
            par = tap % 2
            sl = h1r[:, m:m + 13, par, :].reshape(BN * 13, F)
            o2 = o2 + jnp.dot(sl, w2_ref[tap],
                              preferred_element_type=jnp.float32)
        h2 = _leaky(o2.reshape(BN, 13, F) + b2_ref[0][None, None, :])

        h2p = jnp.concatenate(
            [h2, jnp.zeros((BN, 1, F), jnp.float32)], axis=1)
        h2r = h2p.reshape(BN, 7, 2, F)
        o3 = jnp.zeros((BN * 5, F), jnp.float32)
        for tap in range(5):
            m = tap // 2
            par = tap % 2
            sl = h2r[:, m:m + 5, par, :].reshape(BN * 5, F)
            o3 = o3 + jnp.dot(sl, w3_ref[tap],
                              preferred_element_type=jnp.float32)
        h3 = o3.reshape(BN, 5, F) + b3_ref[0][None, None, :]
        y_ref[0] = jnp.max(h3, axis=1)

    grid = (B, NBLK)
    return pl.pallas_call(
        body,
        grid=grid,
        in_specs=[
            pl.BlockSpec((1, BN, T), lambda b, j: (b, j, 0)),
            pl.BlockSpec((5, 1, F), lambda b, j: (0, 0, 0)),
            pl.BlockSpec((5, F, F), lambda b, j: (0, 0, 0)),
            pl.BlockSpec((5, F, F), lambda b, j: (0, 0, 0)),
            pl.BlockSpec((1, F), lambda b, j: (0, 0)),
            pl.BlockSpec((1, F), lambda b, j: (0, 0)),
            pl.BlockSpec((1, F), lambda b, j: (0, 0)),
        ],
        out_specs=pl.BlockSpec((1, BN, F), lambda b, j: (b, j, 0)),
        out_shape=jax.ShapeDtypeStruct((B, N, F), jnp.float32),
    )(x, w1r, w2t, w3t, b1, b2, b3)


def _tc_scale_z0(y3, dinv3, d2_3):
    def body(y_ref, dinv_ref, d2_ref, za_ref, zb_ref, d2e_ref):
        j = pl.program_id(1)
        dv = dinv_ref[0, 0].reshape(BN, 1)
        rows = jax.lax.broadcasted_iota(jnp.int32, (BN, 1), 0) + j * BN
        z = jnp.where(rows < N, dv * y_ref[0], 0.0)
        za_ref[0] = z[:, :FC]
        zb_ref[0] = z[:, FC:]
        d2e_ref[...] = jnp.broadcast_to(d2_ref[0, 0].reshape(BN, 1), (BN, FC))

    zshape = jax.ShapeDtypeStruct((B, NP, FC), jnp.float32)
    return pl.pallas_call(
        body,
        grid=(B, NBLK),
        in_specs=[
            pl.BlockSpec((1, BN, F), lambda b, j: (b, j, 0)),
            pl.BlockSpec((1, 1, BN), lambda b, j: (j, 0, 0)),
            pl.BlockSpec((1, 1, BN), lambda b, j: (j, 0, 0)),
        ],
        out_specs=[
            pl.BlockSpec((1, BN, FC), lambda b, j: (b, j, 0)),
            pl.BlockSpec((1, BN, FC), lambda b, j: (b, j, 0)),
            pl.BlockSpec((BN, FC), lambda b, j: (j, 0)),
        ],
        out_shape=[zshape, zshape,
                   jax.ShapeDtypeStruct((NP, FC), jnp.float32)],
    )(y3, dinv3, d2_3)


def _tc_mix1(za, zb, sdeg3, dinv3, gw1t, gb1):
    def body(za_ref, zb_ref, sdeg_ref, dinv_ref, w_ref, b_ref,
             oa_ref, ob_ref):
        sv = sdeg_ref[0, 0].reshape(BN, 1)
        dv = dinv_ref[0, 0].reshape(BN, 1)
        x39 = sv * jnp.concatenate([za_ref[0], zb_ref[0]], axis=-1)
        h = _leaky(jnp.dot(x39, w_ref[...],
                           preferred_element_type=jnp.float32)
                   + b_ref[0][None, :])
        zo = dv * h
        oa_ref[0] = zo[:, :FC]
        ob_ref[0] = zo[:, FC:]

    zshape = jax.ShapeDtypeStruct((B, NP, FC), jnp.float32)
    return pl.pallas_call(
        body,
        grid=(B, NBLK),
        in_specs=[
            pl.BlockSpec((1, BN, FC), lambda b, j: (b, j, 0)),
            pl.BlockSpec((1, BN, FC), lambda b, j: (b, j, 0)),
            pl.BlockSpec((1, 1, BN), lambda b, j: (j, 0, 0)),
            pl.BlockSpec((1, 1, BN), lambda b, j: (j, 0, 0)),
            pl.BlockSpec((F, F), lambda b, j: (0, 0)),
            pl.BlockSpec((1, F), lambda b, j: (0, 0)),
        ],
        out_specs=[
            pl.BlockSpec((1, BN, FC), lambda b, j: (b, j, 0)),
            pl.BlockSpec((1, BN, FC), lambda b, j: (b, j, 0)),
        ],
        out_shape=[zshape, zshape],
    )(za, zb, sdeg3, dinv3, gw1t, gb1)


def _tc_mix2(za, zb, sdeg3, gw2t, gb2):
    def body(za_ref, zb_ref, sdeg_ref, w_ref, b_ref, o_ref):
        j = pl.program_id(1)
        sv = sdeg_ref[0, 0].reshape(BN, 1)
        x39 = sv * jnp.concatenate([za_ref[0], zb_ref[0]], axis=-1)
        h = _leaky(jnp.dot(x39, w_ref[...],
                           preferred_element_type=jnp.float32)
                   + b_ref[0][None, :])
        rows = jax.lax.broadcasted_iota(jnp.int32, (BN, 1), 0) + j * BN
        hm = jnp.where(rows < N, h, 0.0)
        ps = jnp.sum(hm.reshape(BN // 8, 8, F), axis=0)[None]

        @pl.when(j == 0)
        def _():
            o_ref[...] = ps

        @pl.when(j != 0)
        def _():
            o_ref[...] = o_ref[...] + ps

    return pl.pallas_call(
        body,
        grid=(B, NBLK),
        in_specs=[
            pl.BlockSpec((1, BN, FC), lambda b, j: (b, j, 0)),
            pl.BlockSpec((1, BN, FC), lambda b, j: (b, j, 0)),
            pl.BlockSpec((1, 1, BN), lambda b, j: (j, 0, 0)),
            pl.BlockSpec((F, F), lambda b, j: (0, 0)),
            pl.BlockSpec((1, F), lambda b, j: (0, 0)),
        ],
        out_specs=pl.BlockSpec((1, 8, F), lambda b, j: (b, 0, 0)),
        out_shape=jax.ShapeDtypeStruct((B, 8, F), jnp.float32),
    )(za, zb, sdeg3, gw2t, gb2)


def _tc_head(hsum, linwt_pad, linb_pad):
    def body(h_ref, w_ref, b_ref, o_ref):
        hs = jnp.sum(h_ref[...], axis=1)
        o_ref[...] = (jnp.dot(hs * (1.0 / N), w_ref[...],
                              preferred_element_type=jnp.float32)
                      + b_ref[...])

    return pl.pallas_call(
        body,
        out_shape=jax.ShapeDtypeStruct((B, F), jnp.float32),
    )(hsum, linwt_pad, linb_pad)


def kernel(x, edge_index, W1, b1, W2, b2, W3, b3, gW1, gb1, gW2, gb2,
           linW, linb):
    E = edge_index.shape[1]
    e_real = E + N
    epad = ((e_real + 32 * EC - 1) // (32 * EC)) * (32 * EC)
    npad_e = epad - e_real

    loop = jnp.arange(N, dtype=jnp.int32)
    pad_idx = N + (jnp.arange(npad_e, dtype=jnp.int32) % (NP - N))
    row_full = jnp.concatenate([edge_index[0], loop, pad_idx])
    col_full = jnp.concatenate([edge_index[1], loop, pad_idx])

    col_a = col_full.reshape(32, epad // (32 * EC), EC)
    row_1 = row_full.reshape(NTEC, epad // (NTEC * EC), EC)
    col_b = col_full.reshape(NTEC, epad // (NTEC * EC), EC)
    row_b = jnp.stack([row_1, row_1 + NP])

    ones_hbm = jnp.ones((EC,), jnp.float32)
    zrow_hbm = jnp.zeros((RPT,), jnp.float32)

    w1r = jnp.transpose(W1, (2, 1, 0))
    w2t = jnp.transpose(W2, (2, 1, 0))
    w3t = jnp.transpose(W3, (2, 1, 0))
    b1r = b1.reshape(1, F)
    b2r = b2.reshape(1, F)
    b3r = b3.reshape(1, F)
    gw1t = jnp.transpose(gW1)
    gw2t = jnp.transpose(gW2)
    linwt_pad = jnp.zeros((F, F), jnp.float32).at[:, :2].set(jnp.transpose(linW))
    linb_pad = jnp.zeros((B, F), jnp.float32).at[:, :2].set(linb[None, :])

    degp = _sc_deg(col_a, ones_hbm, zrow_hbm)
    degp3 = degp.reshape(B, NP // F, F)

    dinv2, sdeg2, d2_2 = _tc_prep(degp3)
    dinv3 = dinv2.reshape(NBLK, 1, BN)
    sdeg3 = sdeg2.reshape(NBLK, 1, BN)
    d2_3 = d2_2.reshape(NBLK, 1, BN)

    y3 = _tc_conv(x, w1r, w2t, w3t, b1r, b2r, b3r)

    z0a, z0b_half, d2e = _tc_scale_z0(y3, dinv3, d2_3)

    za1, zb1 = _sc_hops(z0a.reshape(B * NP, FC), z0b_half.reshape(B * NP, FC),
                        d2e, row_b, col_b)
    ma, mb = _tc_mix1(za1.reshape(B, NP, FC), zb1.reshape(B, NP, FC),
                      sdeg3, dinv3, gw1t, gb1.reshape(1, F))
    za2, zb2 = _sc_hops(ma.reshape(B * NP, FC), mb.reshape(B * NP, FC),
                        d2e, row_b, col_b)
    hsum = _tc_mix2(za2.reshape(B, NP, FC), zb2.reshape(B, NP, FC),
                    sdeg3, gw2t, gb2.reshape(1, F))
    outp = _tc_head(hsum, linwt_pad, linb_pad)

    out = outp[:, :2]
    y = y3.reshape(B * N, F)
    return (out, y)
